# baseline (speedup 1.0000x reference)
import math

import jax
import jax.numpy as jnp
from jax import lax
from jax.experimental import pallas as pl
from jax.experimental.pallas import tpu as pltpu
from jax.experimental.pallas import tpu_sc as plsc

_LS = 0.1
_V = 100000
_B = 1024
_CONF = 1.0 - _LS
_SMOOTH = _LS / (_V - 2)
_C0 = _B * ((_V - 1) * _SMOOTH * math.log(_SMOOTH) + _CONF * math.log(_CONF))

_NW = 32
_RPW = _B // _NW

_LANES = 16
_WCH = 6400
_WTAIL = 3968
_NFULL = 15
_NCH = _NFULL + 1
_CSP = _V - (_V % 128)

_TASKS = [(g, c * _WCH, _WCH if c < _NFULL else _WTAIL)
          for g in range(_RPW // 8) for c in range(_NCH)]


def _tc_kernel(t_ref, x_ref, out_ref):
    sl = x_ref[...]
    scol = _CSP + lax.broadcasted_iota(jnp.int32, sl.shape, 1)
    s_part = jnp.sum(jnp.where(scol < _V, sl, 0.0))
    g_part = jnp.sum(jnp.where(scol == t_ref[...], sl, 0.0))
    out_ref[0] = (_SMOOTH * s_part
                  + (_CONF - _SMOOTH) * g_part).astype(jnp.float32)


def _tc_sliver(output, t2d):
    return pl.pallas_call(
        _tc_kernel,
        grid=(1,),
        in_specs=[
            pl.BlockSpec((_B, 1), lambda j: (0, 0)),
            pl.BlockSpec((_B, 128), lambda j: (0, _CSP // 128)),
        ],
        out_specs=pl.BlockSpec(memory_space=pltpu.SMEM),
        out_shape=jax.ShapeDtypeStruct((1,), jnp.float32),
    )(t2d, output)


def _sc_body(x_hbm, t_hbm, out_hbm, buf_a, buf_b, tgt_v, res_v, sem_a, sem_b):
    wid = lax.axis_index("s") * 2 + lax.axis_index("c")
    rbase = wid * _RPW

    pltpu.sync_copy(t_hbm.at[pl.ds(rbase, _RPW)], tgt_v)
    t_chunks = [tgt_v[pl.ds(q * _LANES, _LANES)]
                for q in range(_RPW // _LANES)]

    bufs = (buf_a, buf_b)
    sems = (sem_a, sem_b)

    def start(ti):
        g, c0, w = _TASKS[ti]
        b = ti % 2
        return pltpu.async_copy(
            x_hbm.at[pl.ds(rbase + g * 8, 8), pl.ds(c0, w)],
            bufs[b].at[pl.ds(0, 8), pl.ds(0, w)], sems[b])

    iota = lax.iota(jnp.int32, _LANES)
    zero = jnp.zeros((_LANES,), jnp.float32)
    acc_s = zero
    acc_g = zero
    pending = start(0)
    for ti in range(len(_TASKS)):
        g, c0, w = _TASKS[ti]
        buf = bufs[ti % 2]
        pending.wait()
        if ti + 1 < len(_TASKS):
            pending = start(ti + 1)

        row_refs = [buf.at[r] for r in range(8)]

        def body(i, accs, row_refs=row_refs):
            return tuple(
                accs[r] + row_refs[r][pl.ds(i * _LANES, _LANES)]
                for r in range(8))

        accs = lax.fori_loop(0, w // _LANES, body, (zero,) * 8)
        for r in range(8):
            acc_s = acc_s + accs[r]
            k = g * 8 + r
            t_r = t_chunks[k // _LANES][k % _LANES]
            rel = t_r - c0
            inb = (rel >= 0) & (rel < w)
            wstart = pl.multiple_of(
                jnp.clip((rel >> 4) << 4, 0, w - _LANES), _LANES)
            gv = row_refs[r][pl.ds(wstart, _LANES)]
            lane = jnp.where(inb, rel & (_LANES - 1), -1)
            acc_g = acc_g + jnp.where(iota == lane, gv, 0.0)

    res_v[...] = _SMOOTH * acc_s + (_CONF - _SMOOTH) * acc_g
    pltpu.sync_copy(res_v, out_hbm.at[pl.ds(wid * _LANES, _LANES)])


_sc_partial = pl.kernel(
    _sc_body,
    mesh=plsc.VectorSubcoreMesh(core_axis_name="c", subcore_axis_name="s"),
    out_type=jax.ShapeDtypeStruct((_NW * _LANES,), jnp.float32),
    scratch_types=[
        pltpu.VMEM((8, _WCH), jnp.float32),
        pltpu.VMEM((8, _WCH), jnp.float32),
        pltpu.VMEM((_RPW,), jnp.int32),
        pltpu.VMEM((_LANES,), jnp.float32),
        pltpu.SemaphoreType.DMA,
        pltpu.SemaphoreType.DMA,
    ],
)


def kernel(output, target):
    t32 = target.astype(jnp.int32)
    sc_out = _sc_partial(output, t32)
    tc_out = _tc_sliver(output, t32.reshape(_B, 1))
    return (_C0 - tc_out[0] - jnp.sum(sc_out)).astype(jnp.float32)


# speedup vs baseline: 1.0046x; 1.0046x over previous
import math

import jax
import jax.numpy as jnp
from jax import lax
from jax.experimental import pallas as pl
from jax.experimental.pallas import tpu as pltpu
from jax.experimental.pallas import tpu_sc as plsc

_LS = 0.1
_V = 100000
_B = 1024
_CONF = 1.0 - _LS
_SMOOTH = _LS / (_V - 2)
_C0 = _B * ((_V - 1) * _SMOOTH * math.log(_SMOOTH) + _CONF * math.log(_CONF))

_NW = 32
_RPW = _B // _NW

_LANES = 16
_WCH = 6400
_WTAIL = 3968
_NFULL = 15
_NCH = _NFULL + 1
_CSP = _V - (_V % 128)

_TASKS = [(g, c * _WCH, _WCH if c < _NFULL else _WTAIL)
          for g in range(_RPW // 8) for c in range(_NCH)]


def _tc_kernel(t_ref, x_ref, out_ref):
    sl = x_ref[...]
    scol = _CSP + lax.broadcasted_iota(jnp.int32, sl.shape, 1)
    s_part = jnp.sum(jnp.where(scol < _V, sl, 0.0))
    g_part = jnp.sum(jnp.where(scol == t_ref[...], sl, 0.0))
    out_ref[0] = (_SMOOTH * s_part
                  + (_CONF - _SMOOTH) * g_part).astype(jnp.float32)


def _tc_sliver(output, t2d):
    return pl.pallas_call(
        _tc_kernel,
        grid=(1,),
        in_specs=[
            pl.BlockSpec((_B, 1), lambda j: (0, 0)),
            pl.BlockSpec((_B, 128), lambda j: (0, _CSP // 128)),
        ],
        out_specs=pl.BlockSpec(memory_space=pltpu.SMEM),
        out_shape=jax.ShapeDtypeStruct((1,), jnp.float32),
    )(t2d, output)


def _sc_body(x_hbm, t_hbm, out_hbm, buf_a, buf_b, tgt_v, res_v, sem_a, sem_b):
    wid = lax.axis_index("s") * 2 + lax.axis_index("c")
    rbase = wid * _RPW

    pltpu.sync_copy(t_hbm.at[pl.ds(rbase, _RPW)], tgt_v)
    t_chunks = [tgt_v[pl.ds(q * _LANES, _LANES)]
                for q in range(_RPW // _LANES)]

    bufs = (buf_a, buf_b)
    sems = (sem_a, sem_b)

    def start(ti):
        g, c0, w = _TASKS[ti]
        b = ti % 2
        return pltpu.async_copy(
            x_hbm.at[pl.ds(rbase + g * 8, 8), pl.ds(c0, w)],
            bufs[b].at[pl.ds(0, 8), pl.ds(0, w)], sems[b])

    iota = lax.iota(jnp.int32, _LANES)
    zero = jnp.zeros((_LANES,), jnp.float32)
    acc_s = zero
    acc_g = zero
    pending = start(0)
    for ti in range(len(_TASKS)):
        g, c0, w = _TASKS[ti]
        buf = bufs[ti % 2]
        pending.wait()
        if ti + 1 < len(_TASKS):
            pending = start(ti + 1)

        row_refs = [buf.at[r] for r in range(8)]

        def body(i, accs, row_refs=row_refs):
            return tuple(
                accs[r] + row_refs[r][pl.ds(i * _LANES, _LANES)]
                for r in range(8))

        accs = lax.fori_loop(0, w // _LANES, body, (zero,) * 8)
        for r in range(8):
            acc_s = acc_s + accs[r]
            k = g * 8 + r
            t_r = t_chunks[k // _LANES][k % _LANES]
            rel = t_r - c0
            inb = (rel >= 0) & (rel < w)
            wstart = pl.multiple_of(
                jnp.clip((rel >> 4) << 4, 0, w - _LANES), _LANES)
            gv = row_refs[r][pl.ds(wstart, _LANES)]
            lane = jnp.where(inb, rel & (_LANES - 1), -1)
            acc_g = acc_g + jnp.where(iota == lane, gv, 0.0)

    res_v[...] = _SMOOTH * acc_s + (_CONF - _SMOOTH) * acc_g
    pltpu.sync_copy(res_v, out_hbm.at[pl.ds(wid * _LANES, _LANES)])


_sc_partial = pl.kernel(
    _sc_body,
    mesh=plsc.VectorSubcoreMesh(core_axis_name="c", subcore_axis_name="s"),
    compiler_params=pltpu.CompilerParams(use_tc_tiling_on_sc=True),
    out_type=jax.ShapeDtypeStruct((_NW * _LANES,), jnp.float32),
    scratch_types=[
        pltpu.VMEM((8, _WCH), jnp.float32),
        pltpu.VMEM((8, _WCH), jnp.float32),
        pltpu.VMEM((_RPW,), jnp.int32),
        pltpu.VMEM((_LANES,), jnp.float32),
        pltpu.SemaphoreType.DMA,
        pltpu.SemaphoreType.DMA,
    ],
)


def kernel(output, target):
    t32 = target.astype(jnp.int32)
    sc_out = _sc_partial(output, t32)
    tc_out = _tc_sliver(output, t32.reshape(_B, 1))
    return (_C0 - tc_out[0] - jnp.sum(sc_out)).astype(jnp.float32)


# speedup vs baseline: 3.6691x; 3.6522x over previous
import math

import jax
import jax.numpy as jnp
from jax import lax
from jax.experimental import pallas as pl
from jax.experimental.pallas import tpu as pltpu
from jax.experimental.pallas import tpu_sc as plsc

_LS = 0.1
_V = 100000
_B = 1024
_CONF = 1.0 - _LS
_SMOOTH = _LS / (_V - 2)
_C0 = _B * ((_V - 1) * _SMOOTH * math.log(_SMOOTH) + _CONF * math.log(_CONF))

_LANES = 16
_NW = 32

_BM = 2048
_SCV = 21 * _BM
_NBLK = (_V - _SCV + _BM - 1) // _BM

_VPW = _SCV // _NW
_WCH = 32
_NCHK = _VPW // _WCH
_GPW = _B // _NW
_GWAVE = 16


def _tc_kernel(x_ref, out_ref, acc_ref):
    j = pl.program_id(0)

    @pl.when(j == 0)
    def _init():
        acc_ref[0] = 0.0

    @pl.when(j < _NBLK - 1)
    def _main():
        acc_ref[0] += jnp.sum(x_ref[...])

    @pl.when(j == _NBLK - 1)
    def _fin():
        x = x_ref[...]
        row = _SCV + j * _BM + lax.broadcasted_iota(jnp.int32, x.shape, 0)
        acc_ref[0] += jnp.sum(jnp.where(row < _V, x, 0.0))
        out_ref[0] = acc_ref[0]


def _tc_partial(xt):
    return pl.pallas_call(
        _tc_kernel,
        grid=(_NBLK,),
        in_specs=[pl.BlockSpec((_BM, _B), lambda j: (_SCV // _BM + j, 0))],
        out_specs=pl.BlockSpec(memory_space=pltpu.SMEM),
        out_shape=jax.ShapeDtypeStruct((1,), jnp.float32),
        scratch_shapes=[pltpu.SMEM((1,), jnp.float32)],
    )(xt)


def _sc_body(x_hbm, t_hbm, out_hbm, buf_a, buf_b, win_v, tgt_v, res_v,
             sem_a, sem_b, gsem):
    wid = lax.axis_index("s") * 2 + lax.axis_index("c")
    iota = lax.iota(jnp.int32, _LANES)
    zero = jnp.zeros((_LANES,), jnp.float32)

    gbase = wid * _GPW
    pltpu.sync_copy(t_hbm.at[pl.ds(gbase, _GPW)], tgt_v)
    t_regs = [tgt_v[pl.ds(q * _LANES, _LANES)] for q in range(_GPW // _LANES)]
    acc_g = zero
    for wave in range(_GPW // _GWAVE):
        copies = []
        for kk in range(_GWAVE):
            k = wave * _GWAVE + kk
            t_k = t_regs[k // _LANES][k % _LANES]
            row0 = pl.multiple_of((t_k >> 3) << 3, 8)
            i = gbase + k
            col0 = pl.multiple_of((i >> 7) << 7, 128)
            copies.append(pltpu.async_copy(
                x_hbm.at[pl.ds(row0, 8), pl.ds(col0, 128)],
                win_v.at[kk], gsem))
        for c in copies:
            c.wait()
        for kk in range(_GWAVE):
            k = wave * _GWAVE + kk
            t_k = t_regs[k // _LANES][k % _LANES]
            t7 = t_k & 7
            i = gbase + k
            sub0 = pl.multiple_of(((i & 127) >> 4) << 4, _LANES)
            lane = i & (_LANES - 1)
            for r8 in range(8):
                chunk = win_v.at[kk].at[r8][pl.ds(sub0, _LANES)]
                sel_lane = jnp.where(t7 == r8, lane, -1)
                acc_g = acc_g + jnp.where(iota == sel_lane, chunk, 0.0)

    wbase = wid * _VPW
    bufs = (buf_a, buf_b)
    sems = (sem_a, sem_b)

    def start(ci):
        return pltpu.async_copy(
            x_hbm.at[pl.ds(pl.multiple_of(wbase + ci * _WCH, 8), _WCH), :],
            bufs[ci % 2], sems[ci % 2])

    accs = [zero] * _LANES
    pending = start(0)
    for ci in range(_NCHK):
        buf = bufs[ci % 2]
        pending.wait()
        if ci + 1 < _NCHK:
            pending = start(ci + 1)
        row_refs = [buf.at[r] for r in range(_WCH)]

        def body(i, a, row_refs=row_refs):
            out = list(a)
            for r in range(_WCH):
                out[r % _LANES] = out[r % _LANES] + row_refs[r][
                    pl.ds(i * _LANES, _LANES)]
            return tuple(out)

        accs = list(lax.fori_loop(0, _B // _LANES, body, tuple(accs)))

    acc_s = accs[0]
    for u in range(1, _LANES):
        acc_s = acc_s + accs[u]

    res_v[pl.ds(0, _LANES)] = acc_s
    res_v[pl.ds(_LANES, _LANES)] = acc_g
    pltpu.sync_copy(res_v, out_hbm.at[pl.ds(wid * 2 * _LANES, 2 * _LANES)])


_sc_partial = pl.kernel(
    _sc_body,
    mesh=plsc.VectorSubcoreMesh(core_axis_name="c", subcore_axis_name="s"),
    out_type=jax.ShapeDtypeStruct((_NW * 2 * _LANES,), jnp.float32),
    scratch_types=[
        pltpu.VMEM((_WCH, _B), jnp.float32),
        pltpu.VMEM((_WCH, _B), jnp.float32),
        pltpu.VMEM((_GWAVE, 8, 128), jnp.float32),
        pltpu.VMEM((_GPW,), jnp.int32),
        pltpu.VMEM((2 * _LANES,), jnp.float32),
        pltpu.SemaphoreType.DMA,
        pltpu.SemaphoreType.DMA,
        pltpu.SemaphoreType.DMA,
    ],
)


def kernel(output, target):
    xt = output.T
    t32 = target.astype(jnp.int32)
    sc_out = _sc_partial(xt, t32)
    tc_out = _tc_partial(xt)
    sc2 = sc_out.reshape(_NW, 2, _LANES)
    s_total = tc_out[0] + jnp.sum(sc2[:, 0, :])
    g_total = jnp.sum(sc2[:, 1, :])
    return (_C0 - _SMOOTH * s_total
            - (_CONF - _SMOOTH) * g_total).astype(jnp.float32)


# speedup vs baseline: 3.8298x; 1.0438x over previous
import math

import jax
import jax.numpy as jnp
from jax import lax
from jax.experimental import pallas as pl
from jax.experimental.pallas import tpu as pltpu
from jax.experimental.pallas import tpu_sc as plsc

_LS = 0.1
_V = 100000
_B = 1024
_CONF = 1.0 - _LS
_SMOOTH = _LS / (_V - 2)
_C0 = _B * ((_V - 1) * _SMOOTH * math.log(_SMOOTH) + _CONF * math.log(_CONF))

_LANES = 16
_NW = 32

_BM = 2048
_SCV = 17 * _BM
_NBLK = (_V - _SCV + _BM - 1) // _BM
_NSL = _BM // 8
_NSL_LAST = (_V - _SCV - (_NBLK - 1) * _BM) // 8

_VPW = _SCV // _NW
_WCH = 32
_NCHK = _VPW // _WCH
_GPW = _B // _NW
_GWAVE = 16


def _block_sum8(x_ref, nsl):
    chains = [x_ref[pl.ds(u * 8, 8), :] for u in range(4)]
    for k in range(4, nsl):
        u = k % 4
        chains[u] = chains[u] + x_ref[pl.ds(k * 8, 8), :]
    return (chains[0] + chains[1]) + (chains[2] + chains[3])


def _tc_kernel(x_ref, out_ref, accv_ref):
    j = pl.program_id(0)

    @pl.when(j == 0)
    def _init():
        accv_ref[...] = jnp.zeros((8, _B), jnp.float32)

    @pl.when(j < _NBLK - 1)
    def _main():
        accv_ref[...] += _block_sum8(x_ref, _NSL)

    @pl.when(j == _NBLK - 1)
    def _fin():
        accv_ref[...] += _block_sum8(x_ref, _NSL_LAST)
        out_ref[0] = jnp.sum(accv_ref[...])


def _tc_partial(xt):
    return pl.pallas_call(
        _tc_kernel,
        grid=(_NBLK,),
        in_specs=[pl.BlockSpec((_BM, _B), lambda j: (_SCV // _BM + j, 0))],
        out_specs=pl.BlockSpec(memory_space=pltpu.SMEM),
        out_shape=jax.ShapeDtypeStruct((1,), jnp.float32),
        scratch_shapes=[pltpu.VMEM((8, _B), jnp.float32)],
    )(xt)


def _sc_body(x_hbm, t_hbm, out_hbm, buf_a, buf_b, win_v, tgt_v, res_v,
             sem_a, sem_b, gsem):
    wid = lax.axis_index("s") * 2 + lax.axis_index("c")
    iota = lax.iota(jnp.int32, _LANES)
    zero = jnp.zeros((_LANES,), jnp.float32)

    gbase = wid * _GPW
    pltpu.sync_copy(t_hbm.at[pl.ds(gbase, _GPW)], tgt_v)
    t_regs = [tgt_v[pl.ds(q * _LANES, _LANES)] for q in range(_GPW // _LANES)]
    acc_g = zero
    for wave in range(_GPW // _GWAVE):
        copies = []
        for kk in range(_GWAVE):
            k = wave * _GWAVE + kk
            t_k = t_regs[k // _LANES][k % _LANES]
            row0 = pl.multiple_of((t_k >> 3) << 3, 8)
            i = gbase + k
            col0 = pl.multiple_of((i >> 7) << 7, 128)
            copies.append(pltpu.async_copy(
                x_hbm.at[pl.ds(row0, 8), pl.ds(col0, 128)],
                win_v.at[kk], gsem))
        for c in copies:
            c.wait()
        for kk in range(_GWAVE):
            k = wave * _GWAVE + kk
            t_k = t_regs[k // _LANES][k % _LANES]
            t7 = t_k & 7
            i = gbase + k
            sub0 = pl.multiple_of(((i & 127) >> 4) << 4, _LANES)
            lane = i & (_LANES - 1)
            for r8 in range(8):
                chunk = win_v.at[kk].at[r8][pl.ds(sub0, _LANES)]
                sel_lane = jnp.where(t7 == r8, lane, -1)
                acc_g = acc_g + jnp.where(iota == sel_lane, chunk, 0.0)

    wbase = wid * _VPW
    bufs = (buf_a, buf_b)
    sems = (sem_a, sem_b)

    def start(ci):
        return pltpu.async_copy(
            x_hbm.at[pl.ds(pl.multiple_of(wbase + ci * _WCH, 8), _WCH), :],
            bufs[ci % 2], sems[ci % 2])

    accs = [zero] * _LANES
    pending = start(0)
    for ci in range(_NCHK):
        buf = bufs[ci % 2]
        pending.wait()
        if ci + 1 < _NCHK:
            pending = start(ci + 1)
        row_refs = [buf.at[r] for r in range(_WCH)]

        def body(i, a, row_refs=row_refs):
            out = list(a)
            for r in range(_WCH):
                out[r % _LANES] = out[r % _LANES] + row_refs[r][
                    pl.ds(i * _LANES, _LANES)]
            return tuple(out)

        accs = list(lax.fori_loop(0, _B // _LANES, body, tuple(accs)))

    acc_s = accs[0]
    for u in range(1, _LANES):
        acc_s = acc_s + accs[u]

    res_v[pl.ds(0, _LANES)] = acc_s
    res_v[pl.ds(_LANES, _LANES)] = acc_g
    pltpu.sync_copy(res_v, out_hbm.at[pl.ds(wid * 2 * _LANES, 2 * _LANES)])


_sc_partial = pl.kernel(
    _sc_body,
    mesh=plsc.VectorSubcoreMesh(core_axis_name="c", subcore_axis_name="s"),
    out_type=jax.ShapeDtypeStruct((_NW * 2 * _LANES,), jnp.float32),
    scratch_types=[
        pltpu.VMEM((_WCH, _B), jnp.float32),
        pltpu.VMEM((_WCH, _B), jnp.float32),
        pltpu.VMEM((_GWAVE, 8, 128), jnp.float32),
        pltpu.VMEM((_GPW,), jnp.int32),
        pltpu.VMEM((2 * _LANES,), jnp.float32),
        pltpu.SemaphoreType.DMA,
        pltpu.SemaphoreType.DMA,
        pltpu.SemaphoreType.DMA,
    ],
)


def kernel(output, target):
    xt = output.T
    t32 = target.astype(jnp.int32)
    sc_out = _sc_partial(xt, t32)
    tc_out = _tc_partial(xt)
    sc2 = sc_out.reshape(_NW, 2, _LANES)
    s_total = tc_out[0] + jnp.sum(sc2[:, 0, :])
    g_total = jnp.sum(sc2[:, 1, :])
    return (_C0 - _SMOOTH * s_total
            - (_CONF - _SMOOTH) * g_total).astype(jnp.float32)


# speedup vs baseline: 3.8305x; 1.0002x over previous
import math

import jax
import jax.numpy as jnp
from jax import lax
from jax.experimental import pallas as pl
from jax.experimental.pallas import tpu as pltpu
from jax.experimental.pallas import tpu_sc as plsc

_LS = 0.1
_V = 100000
_B = 1024
_CONF = 1.0 - _LS
_SMOOTH = _LS / (_V - 2)
_C0 = _B * ((_V - 1) * _SMOOTH * math.log(_SMOOTH) + _CONF * math.log(_CONF))

_LANES = 16
_NW = 32

_BM = 2048
_SCV = 19 * _BM
_NBLK = (_V - _SCV + _BM - 1) // _BM
_NSTEP = _NBLK // 2
_NSL = _BM // 8
_NSL_LAST = (_V - _SCV - (_NBLK - 1) * _BM) // 8

_VPW = _SCV // _NW
_WCH = 32
_NCHK = _VPW // _WCH
_GPW = _B // _NW
_GWAVE = 16


def _block_sum8(x_ref, nsl):
    chains = [x_ref[pl.ds(u * 8, 8), :] for u in range(4)]
    for k in range(4, nsl):
        u = k % 4
        chains[u] = chains[u] + x_ref[pl.ds(k * 8, 8), :]
    return (chains[0] + chains[1]) + (chains[2] + chains[3])


def _tc_kernel(xa_ref, xb_ref, out_ref, accv_ref):
    j = pl.program_id(0)

    @pl.when(j == 0)
    def _init():
        accv_ref[...] = jnp.zeros((8, _B), jnp.float32)

    @pl.when(j < _NSTEP - 1)
    def _main():
        accv_ref[...] += (_block_sum8(xa_ref, _NSL)
                          + _block_sum8(xb_ref, _NSL))

    @pl.when(j == _NSTEP - 1)
    def _fin():
        accv_ref[...] += (_block_sum8(xa_ref, _NSL)
                          + _block_sum8(xb_ref, _NSL_LAST))
        out_ref[0] = jnp.sum(accv_ref[...])


def _tc_partial(xt):
    base = _SCV // _BM
    return pl.pallas_call(
        _tc_kernel,
        grid=(_NSTEP,),
        in_specs=[
            pl.BlockSpec((_BM, _B), lambda j: (base + j, 0)),
            pl.BlockSpec((_BM, _B), lambda j: (base + _NSTEP + j, 0)),
        ],
        out_specs=pl.BlockSpec(memory_space=pltpu.SMEM),
        out_shape=jax.ShapeDtypeStruct((1,), jnp.float32),
        scratch_shapes=[pltpu.VMEM((8, _B), jnp.float32)],
    )(xt, xt)


def _sc_body(x_hbm, t_hbm, out_hbm, buf_a, buf_b, win_v, tgt_v, res_v,
             sem_a, sem_b, gsem):
    wid = lax.axis_index("s") * 2 + lax.axis_index("c")
    iota = lax.iota(jnp.int32, _LANES)
    zero = jnp.zeros((_LANES,), jnp.float32)

    gbase = wid * _GPW
    pltpu.sync_copy(t_hbm.at[pl.ds(gbase, _GPW)], tgt_v)
    t_regs = [tgt_v[pl.ds(q * _LANES, _LANES)] for q in range(_GPW // _LANES)]
    acc_g = zero
    for wave in range(_GPW // _GWAVE):
        copies = []
        for kk in range(_GWAVE):
            k = wave * _GWAVE + kk
            t_k = t_regs[k // _LANES][k % _LANES]
            row0 = pl.multiple_of((t_k >> 3) << 3, 8)
            i = gbase + k
            col0 = pl.multiple_of((i >> 7) << 7, 128)
            copies.append(pltpu.async_copy(
                x_hbm.at[pl.ds(row0, 8), pl.ds(col0, 128)],
                win_v.at[kk], gsem))
        for c in copies:
            c.wait()
        for kk in range(_GWAVE):
            k = wave * _GWAVE + kk
            t_k = t_regs[k // _LANES][k % _LANES]
            t7 = t_k & 7
            i = gbase + k
            sub0 = pl.multiple_of(((i & 127) >> 4) << 4, _LANES)
            lane = i & (_LANES - 1)
            for r8 in range(8):
                chunk = win_v.at[kk].at[r8][pl.ds(sub0, _LANES)]
                sel_lane = jnp.where(t7 == r8, lane, -1)
                acc_g = acc_g + jnp.where(iota == sel_lane, chunk, 0.0)

    wbase = wid * _VPW
    bufs = (buf_a, buf_b)
    sems = (sem_a, sem_b)

    def start(ci):
        return pltpu.async_copy(
            x_hbm.at[pl.ds(pl.multiple_of(wbase + ci * _WCH, 8), _WCH), :],
            bufs[ci % 2], sems[ci % 2])

    accs = [zero] * _LANES
    pending = start(0)
    for ci in range(_NCHK):
        buf = bufs[ci % 2]
        pending.wait()
        if ci + 1 < _NCHK:
            pending = start(ci + 1)
        row_refs = [buf.at[r] for r in range(_WCH)]

        def body(i, a, row_refs=row_refs):
            out = list(a)
            for r in range(_WCH):
                out[r % _LANES] = out[r % _LANES] + row_refs[r][
                    pl.ds(i * _LANES, _LANES)]
            return tuple(out)

        accs = list(lax.fori_loop(0, _B // _LANES, body, tuple(accs)))

    acc_s = accs[0]
    for u in range(1, _LANES):
        acc_s = acc_s + accs[u]

    res_v[pl.ds(0, _LANES)] = acc_s
    res_v[pl.ds(_LANES, _LANES)] = acc_g
    pltpu.sync_copy(res_v, out_hbm.at[pl.ds(wid * 2 * _LANES, 2 * _LANES)])


_sc_partial = pl.kernel(
    _sc_body,
    mesh=plsc.VectorSubcoreMesh(core_axis_name="c", subcore_axis_name="s"),
    out_type=jax.ShapeDtypeStruct((_NW * 2 * _LANES,), jnp.float32),
    scratch_types=[
        pltpu.VMEM((_WCH, _B), jnp.float32),
        pltpu.VMEM((_WCH, _B), jnp.float32),
        pltpu.VMEM((_GWAVE, 8, 128), jnp.float32),
        pltpu.VMEM((_GPW,), jnp.int32),
        pltpu.VMEM((2 * _LANES,), jnp.float32),
        pltpu.SemaphoreType.DMA,
        pltpu.SemaphoreType.DMA,
        pltpu.SemaphoreType.DMA,
    ],
)


def kernel(output, target):
    xt = output.T
    t32 = target.astype(jnp.int32)
    sc_out = _sc_partial(xt, t32)
    tc_out = _tc_partial(xt)
    sc2 = sc_out.reshape(_NW, 2, _LANES)
    s_total = tc_out[0] + jnp.sum(sc2[:, 0, :])
    g_total = jnp.sum(sc2[:, 1, :])
    return (_C0 - _SMOOTH * s_total
            - (_CONF - _SMOOTH) * g_total).astype(jnp.float32)
